# baseline (speedup 1.0000x reference)
import jax
import jax.numpy as jnp
from jax.experimental import pallas as pl

N = 50000
NPAD = 51200
F_IN = 128
SEQ_LEN = 12
H = 32
G = 4
BN = 2048
BP = BN // G
NBLK = NPAD // BN
HG = H * G
W4 = 4 * HG


def _lstm_head_kernel(xt_ref, wbd_ref, k0_ref, wh0_ref, w1_ref, k1_ref,
                      wf1_ref, bf1_ref, wf2_ref, bf2_ref, y_ref):
    bf16 = jnp.bfloat16
    xt = xt_ref[...]
    k0 = k0_ref[...]
    wh0 = wh0_ref[...]
    w1 = w1_ref[...]
    k1 = k1_ref[...]

    gin = jnp.dot(xt, wbd_ref[...], preferred_element_type=jnp.float32)

    zeros = jnp.zeros((xt.shape[0], HG), dtype=jnp.float32)
    h0, c0, h1, c1 = zeros, zeros, zeros, zeros

    for j in range(SEQ_LEN):
        g = (gin[:, j * W4:(j + 1) * W4] + k0
             + jnp.dot(h0.astype(bf16), wh0,
                       preferred_element_type=jnp.float32))
        a = jnp.tanh(g)
        si = 0.5 * a[:, 0:HG] + 0.5
        sf = 0.5 * a[:, HG:2 * HG] + 0.5
        so = 0.5 * a[:, 3 * HG:4 * HG] + 0.5
        c0 = sf * c0 + si * a[:, 2 * HG:3 * HG]
        h0 = so * jnp.tanh(c0)

        hcat = jnp.concatenate([h0.astype(bf16), h1.astype(bf16)], axis=1)
        g1 = jnp.dot(hcat, w1, preferred_element_type=jnp.float32) + k1
        a1 = jnp.tanh(g1)
        si1 = 0.5 * a1[:, 0:HG] + 0.5
        sf1 = 0.5 * a1[:, HG:2 * HG] + 0.5
        so1 = 0.5 * a1[:, 3 * HG:4 * HG] + 0.5
        c1 = sf1 * c1 + si1 * a1[:, 2 * HG:3 * HG]
        h1 = so1 * jnp.tanh(c1)

    z = jax.nn.relu(
        jnp.dot(h1.astype(bf16), wf1_ref[...],
                preferred_element_type=jnp.float32)
        + bf1_ref[...])
    y = jnp.dot(z.astype(bf16), wf2_ref[...],
                preferred_element_type=jnp.float32)
    y_ref[...] = y + bf2_ref[...]


def kernel(x, edge_index, W_fp, b_fp, W1, b1, W2, b2, W3, b3, W_tp, b_tp,
           W_ih0, W_hh0, b_ih0, b_hh0, W_ih1, W_hh1, b_ih1, b_hh1,
           W_f1, b_f1, W_f2, b_f2):
    f32 = jnp.float32
    x_tail = jax.lax.slice(x, (0, F_IN - SEQ_LEN), (N, F_IN))
    x_tail = jnp.pad(x_tail, ((0, NPAD - N), (0, 0)))

    xp = x_tail.reshape(NPAD // G, SEQ_LEN * G).astype(jnp.bfloat16)

    I4 = jnp.eye(G, dtype=f32)
    I12 = jnp.eye(SEQ_LEN, dtype=f32)
    half32 = jnp.full((H,), 0.5, f32)
    one32 = jnp.ones((H,), f32)
    sc = jnp.concatenate([half32, half32, one32, half32])[None, :]

    v0 = ((W_tp @ W_ih0.T) * sc).reshape(4, H)
    k0 = (((b_tp @ W_ih0.T + b_ih0 + b_hh0)[None, :]) * sc)
    k0big = jnp.broadcast_to(k0.reshape(4, 1, H),
                             (4, G, H)).reshape(1, W4)
    wbd = jnp.einsum('jk,gh,bu->gjkbhu', I12, I4, v0).reshape(
        SEQ_LEN * G, SEQ_LEN * W4)

    wh0s = (W_hh0.T * sc).reshape(H, 4, H)
    wh0b = jnp.einsum('gh,kbu->gkbhu', I4, wh0s).reshape(HG, W4)

    w1s = (jnp.concatenate([W_ih1, W_hh1], axis=1).T * sc)
    w1a = w1s[0:H].reshape(H, 4, H)
    w1b = w1s[H:2 * H].reshape(H, 4, H)
    w1big = jnp.concatenate([
        jnp.einsum('gh,kbu->gkbhu', I4, w1a).reshape(HG, W4),
        jnp.einsum('gh,kbu->gkbhu', I4, w1b).reshape(HG, W4),
    ], axis=0)
    k1 = (((b_ih1 + b_hh1)[None, :]) * sc)
    k1big = jnp.broadcast_to(k1.reshape(4, 1, H),
                             (4, G, H)).reshape(1, W4)

    wf1b = jnp.einsum('gh,ku->gkhu', I4, W_f1).reshape(HG, 16 * G)
    bf1b = jnp.broadcast_to(b_f1[None, None, :], (1, G, 16)).reshape(1, 16 * G)
    wf2b = jnp.einsum('gh,u->guh', I4, W_f2[:, 0]).reshape(16 * G, G)
    bf2b = b_f2[None, :]

    bf16 = jnp.bfloat16
    wbd = wbd.astype(bf16)
    wh0b = wh0b.astype(bf16)
    w1big = w1big.astype(bf16)
    wf1b = wf1b.astype(bf16)
    wf2b = wf2b.astype(bf16)

    full = lambda i: (0, 0)
    yp = pl.pallas_call(
        _lstm_head_kernel,
        grid=(NBLK,),
        in_specs=[
            pl.BlockSpec((BP, SEQ_LEN * G), lambda i: (i, 0)),
            pl.BlockSpec(wbd.shape, full),
            pl.BlockSpec(k0big.shape, full),
            pl.BlockSpec(wh0b.shape, full),
            pl.BlockSpec(w1big.shape, full),
            pl.BlockSpec(k1big.shape, full),
            pl.BlockSpec(wf1b.shape, full),
            pl.BlockSpec(bf1b.shape, full),
            pl.BlockSpec(wf2b.shape, full),
            pl.BlockSpec(bf2b.shape, full),
        ],
        out_specs=pl.BlockSpec((BP, G), lambda i: (i, 0)),
        out_shape=jax.ShapeDtypeStruct((NPAD // G, G), f32),
    )(xp, wbd, k0big, wh0b, w1big, k1big, wf1b, bf1b, wf2b, bf2b)

    y = yp.reshape(NPAD, 1)
    return jax.lax.slice(y, (0, 0), (N, 1))


# speedup vs baseline: 1.0320x; 1.0320x over previous
import jax
import jax.numpy as jnp
from jax.experimental import pallas as pl

N = 50000
NPAD = 51200
F_IN = 128
SEQ_LEN = 12
H = 32
G = 4
BN = 5120
BP = BN // G
NBLK = NPAD // BN
HG = H * G
W4 = 4 * HG


def _lstm_head_kernel(xt_ref, wbd_ref, k0_ref, wh0_ref, w1_ref, k1_ref,
                      wf1_ref, bf1_ref, wf2_ref, bf2_ref, y_ref):
    bf16 = jnp.bfloat16
    xt = xt_ref[...]
    k0 = k0_ref[...]
    wh0 = wh0_ref[...]
    w1 = w1_ref[...]
    k1 = k1_ref[...]

    gin = jnp.dot(xt, wbd_ref[...], preferred_element_type=jnp.float32)

    zeros = jnp.zeros((xt.shape[0], HG), dtype=jnp.float32)
    h0, c0, h1, c1 = zeros, zeros, zeros, zeros

    for j in range(SEQ_LEN):
        g = (gin[:, j * W4:(j + 1) * W4] + k0
             + jnp.dot(h0.astype(bf16), wh0,
                       preferred_element_type=jnp.float32))
        a = jnp.tanh(g)
        si = 0.5 * a[:, 0:HG] + 0.5
        sf = 0.5 * a[:, HG:2 * HG] + 0.5
        so = 0.5 * a[:, 3 * HG:4 * HG] + 0.5
        c0 = sf * c0 + si * a[:, 2 * HG:3 * HG]
        h0 = so * jnp.tanh(c0)

        hcat = jnp.concatenate([h0.astype(bf16), h1.astype(bf16)], axis=1)
        g1 = jnp.dot(hcat, w1, preferred_element_type=jnp.float32) + k1
        a1 = jnp.tanh(g1)
        si1 = 0.5 * a1[:, 0:HG] + 0.5
        sf1 = 0.5 * a1[:, HG:2 * HG] + 0.5
        so1 = 0.5 * a1[:, 3 * HG:4 * HG] + 0.5
        c1 = sf1 * c1 + si1 * a1[:, 2 * HG:3 * HG]
        h1 = so1 * jnp.tanh(c1)

    z = jax.nn.relu(
        jnp.dot(h1.astype(bf16), wf1_ref[...],
                preferred_element_type=jnp.float32)
        + bf1_ref[...])
    y = jnp.dot(z.astype(bf16), wf2_ref[...],
                preferred_element_type=jnp.float32)
    y_ref[...] = y + bf2_ref[...]


def kernel(x, edge_index, W_fp, b_fp, W1, b1, W2, b2, W3, b3, W_tp, b_tp,
           W_ih0, W_hh0, b_ih0, b_hh0, W_ih1, W_hh1, b_ih1, b_hh1,
           W_f1, b_f1, W_f2, b_f2):
    f32 = jnp.float32
    x_tail = jax.lax.slice(x, (0, F_IN - SEQ_LEN), (N, F_IN))
    x_tail = jnp.pad(x_tail, ((0, NPAD - N), (0, 0)))

    xp = x_tail.reshape(NPAD // G, SEQ_LEN * G).astype(jnp.bfloat16)

    I4 = jnp.eye(G, dtype=f32)
    I12 = jnp.eye(SEQ_LEN, dtype=f32)
    half32 = jnp.full((H,), 0.5, f32)
    one32 = jnp.ones((H,), f32)
    sc = jnp.concatenate([half32, half32, one32, half32])[None, :]

    v0 = ((W_tp @ W_ih0.T) * sc).reshape(4, H)
    k0 = (((b_tp @ W_ih0.T + b_ih0 + b_hh0)[None, :]) * sc)
    k0big = jnp.broadcast_to(k0.reshape(4, 1, H),
                             (4, G, H)).reshape(1, W4)
    wbd = jnp.einsum('jk,gh,bu->gjkbhu', I12, I4, v0).reshape(
        SEQ_LEN * G, SEQ_LEN * W4)

    wh0s = (W_hh0.T * sc).reshape(H, 4, H)
    wh0b = jnp.einsum('gh,kbu->gkbhu', I4, wh0s).reshape(HG, W4)

    w1s = (jnp.concatenate([W_ih1, W_hh1], axis=1).T * sc)
    w1a = w1s[0:H].reshape(H, 4, H)
    w1b = w1s[H:2 * H].reshape(H, 4, H)
    w1big = jnp.concatenate([
        jnp.einsum('gh,kbu->gkbhu', I4, w1a).reshape(HG, W4),
        jnp.einsum('gh,kbu->gkbhu', I4, w1b).reshape(HG, W4),
    ], axis=0)
    k1 = (((b_ih1 + b_hh1)[None, :]) * sc)
    k1big = jnp.broadcast_to(k1.reshape(4, 1, H),
                             (4, G, H)).reshape(1, W4)

    wf1b = jnp.einsum('gh,ku->gkhu', I4, W_f1).reshape(HG, 16 * G)
    bf1b = jnp.broadcast_to(b_f1[None, None, :], (1, G, 16)).reshape(1, 16 * G)
    wf2b = jnp.einsum('gh,u->guh', I4, W_f2[:, 0]).reshape(16 * G, G)
    bf2b = b_f2[None, :]

    bf16 = jnp.bfloat16
    wbd = wbd.astype(bf16)
    wh0b = wh0b.astype(bf16)
    w1big = w1big.astype(bf16)
    wf1b = wf1b.astype(bf16)
    wf2b = wf2b.astype(bf16)

    full = lambda i: (0, 0)
    yp = pl.pallas_call(
        _lstm_head_kernel,
        grid=(NBLK,),
        in_specs=[
            pl.BlockSpec((BP, SEQ_LEN * G), lambda i: (i, 0)),
            pl.BlockSpec(wbd.shape, full),
            pl.BlockSpec(k0big.shape, full),
            pl.BlockSpec(wh0b.shape, full),
            pl.BlockSpec(w1big.shape, full),
            pl.BlockSpec(k1big.shape, full),
            pl.BlockSpec(wf1b.shape, full),
            pl.BlockSpec(bf1b.shape, full),
            pl.BlockSpec(wf2b.shape, full),
            pl.BlockSpec(bf2b.shape, full),
        ],
        out_specs=pl.BlockSpec((BP, G), lambda i: (i, 0)),
        out_shape=jax.ShapeDtypeStruct((NPAD // G, G), f32),
    )(xp, wbd, k0big, wh0b, w1big, k1big, wf1b, bf1b, wf2b, bf2b)

    y = yp.reshape(NPAD, 1)
    return jax.lax.slice(y, (0, 0), (N, 1))


# speedup vs baseline: 1.0634x; 1.0304x over previous
import jax
import jax.numpy as jnp
from jax.experimental import pallas as pl

N = 50000
F_IN = 128
SEQ_LEN = 12
H = 32
G = 4
NP = N // G
BP = 1280
HG = H * G
W4 = 4 * HG
CS = F_IN - 16


def _lstm_head_kernel(xr_ref, wbd_ref, k0_ref, wh0_ref, w1_ref, k1_ref,
                      wf1_ref, bf1_ref, wf2_ref, bf2_ref, y_ref):
    bf16 = jnp.bfloat16
    xr = xr_ref[...]
    k0 = k0_ref[...]
    wh0 = wh0_ref[...]
    w1 = w1_ref[...]
    k1 = k1_ref[...]

    xt = jnp.concatenate(
        [xr[:, g * F_IN + CS:g * F_IN + CS + 16] for g in range(G)],
        axis=1).astype(bf16)

    gin = jnp.dot(xt, wbd_ref[...], preferred_element_type=jnp.float32)

    zeros = jnp.zeros((xt.shape[0], HG), dtype=jnp.float32)
    h0, c0, h1, c1 = zeros, zeros, zeros, zeros

    for j in range(SEQ_LEN):
        g = (gin[:, j * W4:(j + 1) * W4] + k0
             + jnp.dot(h0.astype(bf16), wh0,
                       preferred_element_type=jnp.float32))
        a = jnp.tanh(g)
        si = 0.5 * a[:, 0:HG] + 0.5
        sf = 0.5 * a[:, HG:2 * HG] + 0.5
        so = 0.5 * a[:, 3 * HG:4 * HG] + 0.5
        c0 = sf * c0 + si * a[:, 2 * HG:3 * HG]
        h0 = so * jnp.tanh(c0)

        hcat = jnp.concatenate([h0.astype(bf16), h1.astype(bf16)], axis=1)
        g1 = jnp.dot(hcat, w1, preferred_element_type=jnp.float32) + k1
        a1 = jnp.tanh(g1)
        si1 = 0.5 * a1[:, 0:HG] + 0.5
        sf1 = 0.5 * a1[:, HG:2 * HG] + 0.5
        so1 = 0.5 * a1[:, 3 * HG:4 * HG] + 0.5
        c1 = sf1 * c1 + si1 * a1[:, 2 * HG:3 * HG]
        h1 = so1 * jnp.tanh(c1)

    z = jax.nn.relu(
        jnp.dot(h1.astype(bf16), wf1_ref[...],
                preferred_element_type=jnp.float32)
        + bf1_ref[...])
    y = jnp.dot(z.astype(bf16), wf2_ref[...],
                preferred_element_type=jnp.float32)
    y_ref[...] = y + bf2_ref[...]


def kernel(x, edge_index, W_fp, b_fp, W1, b1, W2, b2, W3, b3, W_tp, b_tp,
           W_ih0, W_hh0, b_ih0, b_hh0, W_ih1, W_hh1, b_ih1, b_hh1,
           W_f1, b_f1, W_f2, b_f2):
    f32 = jnp.float32
    bf16 = jnp.bfloat16
    xr = x.reshape(NP, G * F_IN)

    I4 = jnp.eye(G, dtype=f32)
    I12 = jnp.eye(SEQ_LEN, dtype=f32)
    half32 = jnp.full((H,), 0.5, f32)
    one32 = jnp.ones((H,), f32)
    sc = jnp.concatenate([half32, half32, one32, half32])[None, :]

    v0 = ((W_tp @ W_ih0.T) * sc).reshape(4, H)
    k0 = (((b_tp @ W_ih0.T + b_ih0 + b_hh0)[None, :]) * sc)
    k0big = jnp.broadcast_to(k0.reshape(4, 1, H),
                             (4, G, H)).reshape(1, W4)
    wbd = jnp.einsum('jk,gh,bu->gjkbhu', I12, I4, v0).reshape(
        SEQ_LEN * G, SEQ_LEN * W4)
    wbd = wbd.reshape(G, SEQ_LEN, SEQ_LEN * W4)
    wbd = jnp.pad(wbd, ((0, 0), (4, 0), (0, 0))).reshape(
        G * 16, SEQ_LEN * W4)

    wh0s = (W_hh0.T * sc).reshape(H, 4, H)
    wh0b = jnp.einsum('gh,kbu->gkbhu', I4, wh0s).reshape(HG, W4)

    w1s = (jnp.concatenate([W_ih1, W_hh1], axis=1).T * sc)
    w1a = w1s[0:H].reshape(H, 4, H)
    w1b = w1s[H:2 * H].reshape(H, 4, H)
    w1big = jnp.concatenate([
        jnp.einsum('gh,kbu->gkbhu', I4, w1a).reshape(HG, W4),
        jnp.einsum('gh,kbu->gkbhu', I4, w1b).reshape(HG, W4),
    ], axis=0)
    k1 = (((b_ih1 + b_hh1)[None, :]) * sc)
    k1big = jnp.broadcast_to(k1.reshape(4, 1, H),
                             (4, G, H)).reshape(1, W4)

    wf1b = jnp.einsum('gh,ku->gkhu', I4, W_f1).reshape(HG, 16 * G)
    bf1b = jnp.broadcast_to(b_f1[None, None, :], (1, G, 16)).reshape(1, 16 * G)
    wf2b = jnp.einsum('gh,u->guh', I4, W_f2[:, 0]).reshape(16 * G, G)
    bf2b = b_f2[None, :]

    wbd = wbd.astype(bf16)
    wh0b = wh0b.astype(bf16)
    w1big = w1big.astype(bf16)
    wf1b = wf1b.astype(bf16)
    wf2b = wf2b.astype(bf16)

    full = lambda i: (0, 0)
    yp = pl.pallas_call(
        _lstm_head_kernel,
        grid=(pl.cdiv(NP, BP),),
        in_specs=[
            pl.BlockSpec((BP, G * F_IN), lambda i: (i, 0)),
            pl.BlockSpec(wbd.shape, full),
            pl.BlockSpec(k0big.shape, full),
            pl.BlockSpec(wh0b.shape, full),
            pl.BlockSpec(w1big.shape, full),
            pl.BlockSpec(k1big.shape, full),
            pl.BlockSpec(wf1b.shape, full),
            pl.BlockSpec(bf1b.shape, full),
            pl.BlockSpec(wf2b.shape, full),
            pl.BlockSpec(bf2b.shape, full),
        ],
        out_specs=pl.BlockSpec((BP, G), lambda i: (i, 0)),
        out_shape=jax.ShapeDtypeStruct((NP, G), f32),
    )(xr, wbd, k0big, wh0b, w1big, k1big, wf1b, bf1b, wf2b, bf2b)

    return yp.reshape(N, 1)
